# Initial kernel scaffold; baseline (speedup 1.0000x reference)
import jax
import jax.numpy as jnp
from jax.experimental import pallas as pl


def kernel(adj_bin, recip_deg, feat, w_self_0, w_neigh_0, bias_0, w_self_1, w_neigh_1, bias_1, w_self_2, w_neigh_2, bias_2):
    raise NotImplementedError("write your pallas kernel here")


# speedup vs baseline: 2.7562x; 2.7562x over previous
import jax
import jax.numpy as jnp
from jax.experimental import pallas as pl
from jax.experimental.pallas import tpu as pltpu

_N = 4000
_N_PAD = 4096
_TILE = 256
_LANE = 128


def _round_up(x, m):
    return ((x + m - 1) // m) * m


def _proj_kernel(h_ref, w_ref, hp_ref):
    hp = jnp.dot(h_ref[...], w_ref[...], preferred_element_type=jnp.float32)
    hp_ref[...] = hp.astype(hp_ref.dtype)


def _project(h, w):
    n_pad, d_in = h.shape
    d_out = w.shape[1]
    blk = n_pad // 2
    return pl.pallas_call(
        _proj_kernel,
        out_shape=jax.ShapeDtypeStruct((n_pad, d_out), jnp.bfloat16),
        grid=(2,),
        in_specs=[
            pl.BlockSpec((blk, d_in), lambda i: (i, 0)),
            pl.BlockSpec((d_in, d_out), lambda i: (0, 0)),
        ],
        out_specs=pl.BlockSpec((blk, d_out), lambda i: (i, 0)),
        compiler_params=pltpu.CompilerParams(dimension_semantics=("parallel",)),
    )(h, w)


def _agg_stats_kernel(a_ref, hp_ref, h_ref, ws_ref, b_ref, rd_ref,
                      out_ref, sum_ref, sq_ref):
    a = a_ref[...].astype(jnp.bfloat16)
    neigh = jnp.dot(a, hp_ref[...], preferred_element_type=jnp.float32)
    self_p = jnp.dot(h_ref[...], ws_ref[...], preferred_element_type=jnp.float32)
    out = neigh * rd_ref[...] + self_p + b_ref[...]
    outc = out.astype(out_ref.dtype)
    out_ref[...] = outc

    i = pl.program_id(0)
    rows = i * _TILE + jax.lax.broadcasted_iota(jnp.int32, outc.shape, 0)
    x = jnp.where(rows < _N, outc.astype(jnp.float32), 0.0)
    sum_ref[0, 0, :] = jnp.sum(x, axis=0)
    sq_ref[0, 0, :] = jnp.sum(x * x, axis=0)


def _agg_plain_kernel(a_ref, hp_ref, h_ref, ws_ref, b_ref, rd_ref, out_ref):
    a = a_ref[...].astype(jnp.bfloat16)
    neigh = jnp.dot(a, hp_ref[...], preferred_element_type=jnp.float32)
    self_p = jnp.dot(h_ref[...], ws_ref[...], preferred_element_type=jnp.float32)
    out = neigh * rd_ref[...] + self_p + b_ref[...]
    out_ref[...] = out.astype(out_ref.dtype)


def _aggregate(a8, hp, h, ws, b, rd, *, out_dtype, with_stats):
    n_pad = a8.shape[0]
    d_in = h.shape[1]
    d_out = hp.shape[1]
    grid = (n_pad // _TILE,)

    out_shapes = [jax.ShapeDtypeStruct((n_pad, d_out), out_dtype)]
    out_specs = [pl.BlockSpec((_TILE, d_out), lambda i: (i, 0))]
    if with_stats:
        nt = n_pad // _TILE
        out_shapes += [jax.ShapeDtypeStruct((nt, 1, d_out), jnp.float32)] * 2
        out_specs += [pl.BlockSpec((1, 1, d_out), lambda i: (i, 0, 0))] * 2

    res = pl.pallas_call(
        _agg_stats_kernel if with_stats else _agg_plain_kernel,
        out_shape=tuple(out_shapes) if with_stats else out_shapes[0],
        grid=grid,
        in_specs=[
            pl.BlockSpec((_TILE, n_pad), lambda i: (i, 0)),
            pl.BlockSpec((n_pad, d_out), lambda i: (0, 0)),
            pl.BlockSpec((_TILE, d_in), lambda i: (i, 0)),
            pl.BlockSpec((d_in, d_out), lambda i: (0, 0)),
            pl.BlockSpec((1, d_out), lambda i: (0, 0)),
            pl.BlockSpec((_TILE, 1), lambda i: (i, 0)),
        ],
        out_specs=tuple(out_specs) if with_stats else out_specs[0],
        compiler_params=pltpu.CompilerParams(
            dimension_semantics=("parallel",)),
    )(a8, hp, h, ws, b, rd)
    return res


def _bnproj_kernel(x_ref, sum_ref, sq_ref, wn_ref, h_ref, hp_ref):
    inv_n = jnp.float32(1.0 / _N)
    s = jnp.sum(sum_ref[...], axis=0)
    sq = jnp.sum(sq_ref[...], axis=0)
    mean = s * inv_n
    var = sq * inv_n - mean * mean
    rstd = jax.lax.rsqrt(var + 1e-5)
    x = x_ref[...].astype(jnp.float32)
    hn = jnp.maximum((x - mean) * rstd, 0.0).astype(jnp.bfloat16)
    h_ref[...] = hn
    hp = jnp.dot(hn, wn_ref[...], preferred_element_type=jnp.float32)
    hp_ref[...] = hp.astype(jnp.bfloat16)


def _bn_relu_project(x, s, sq, wn):
    n_pad, d = x.shape
    d_out = wn.shape[1]
    nt = n_pad // _TILE
    return pl.pallas_call(
        _bnproj_kernel,
        out_shape=(jax.ShapeDtypeStruct((n_pad, d), jnp.bfloat16),
                   jax.ShapeDtypeStruct((n_pad, d_out), jnp.bfloat16)),
        grid=(nt,),
        in_specs=[
            pl.BlockSpec((_TILE, d), lambda i: (i, 0)),
            pl.BlockSpec((nt, 1, d), lambda i: (0, 0, 0)),
            pl.BlockSpec((nt, 1, d), lambda i: (0, 0, 0)),
            pl.BlockSpec((d, d_out), lambda i: (0, 0)),
        ],
        out_specs=(pl.BlockSpec((_TILE, d), lambda i: (i, 0)),
                   pl.BlockSpec((_TILE, d_out), lambda i: (i, 0))),
        compiler_params=pltpu.CompilerParams(dimension_semantics=("parallel",)),
    )(x, s, sq, wn)


def kernel(adj_bin, recip_deg, feat,
           w_self_0, w_neigh_0, bias_0,
           w_self_1, w_neigh_1, bias_1,
           w_self_2, w_neigh_2, bias_2):
    n, d_in = feat.shape
    n_pad = _N_PAD

    a8 = jnp.pad(adj_bin.astype(jnp.float8_e4m3fn),
                 ((0, n_pad - n), (0, n_pad - n)))
    rd = jnp.pad(recip_deg, ((0, n_pad - n), (0, 0))).astype(jnp.float32)
    h = jnp.pad(feat, ((0, n_pad - n), (0, _round_up(d_in, _LANE) - d_in))
                ).astype(jnp.bfloat16)

    params = [(w_self_0, w_neigh_0, bias_0),
              (w_self_1, w_neigh_1, bias_1),
              (w_self_2, w_neigh_2, bias_2)]

    def prep(w):
        d_i, d_o = w.shape
        return jnp.pad(w, ((0, _round_up(d_i, _LANE) - d_i),
                           (0, _round_up(d_o, _LANE) - d_o))).astype(jnp.bfloat16)

    def prep_b(b):
        d_o = b.shape[1]
        return jnp.pad(b, ((0, 0), (0, _round_up(d_o, _LANE) - d_o))
                       ).astype(jnp.float32)

    n_layers = len(params)
    hp = _project(h, prep(params[0][1]))
    for i, (w_self, w_neigh, bias) in enumerate(params):
        last = (i == n_layers - 1)
        res = _aggregate(a8, hp, h, prep(w_self), prep_b(bias), rd,
                         out_dtype=jnp.float32 if last else jnp.bfloat16,
                         with_stats=not last)
        if not last:
            out, s, sq = res
            h, hp = _bn_relu_project(out, s, sq, prep(params[i + 1][1]))
        else:
            out = res

    n_classes = params[-1][0].shape[1]
    return out[:n, :n_classes]


# speedup vs baseline: 4.7687x; 1.7302x over previous
import jax
import jax.numpy as jnp
from jax.experimental import pallas as pl
from jax.experimental.pallas import tpu as pltpu

_N = 4000
_T = 256
_NT = 16
_N_PAD = _NT * _T
_LANE = 128


def _round_up(x, m):
    return ((x + m - 1) // m) * m


def _mega_kernel(a_ref, feat_ref, rd_ref,
                 wn0_ref, ws0_ref, b0_ref,
                 wn1_ref, ws1_ref, b1_ref,
                 wn2_ref, ws2_ref, b2_ref,
                 out_ref,
                 a8_s, hp_s, h_s, x_s, s_s, q_s):
    i = pl.program_id(0)
    inv_n = jnp.float32(1.0 / _N)

    @pl.when(i == 0)
    def _():
        s_s[...] = jnp.zeros_like(s_s)
        q_s[...] = jnp.zeros_like(q_s)
        hp = jnp.dot(feat_ref[...], wn0_ref[...],
                     preferred_element_type=jnp.float32)
        hp_s[...] = hp.astype(jnp.bfloat16)

    a = a_ref[...]
    rows_a = i * _T + jax.lax.broadcasted_iota(jnp.int32, a.shape, 0)
    ab = jnp.where(rows_a < _N, a, 0.0).astype(jnp.bfloat16)
    a8_s[pl.ds(i * _T, _T), :] = ab.astype(jnp.float8_e4m3fn)

    off = pl.ds(i * _T, _T)
    neigh = jnp.dot(ab, hp_s[0:_N, :], preferred_element_type=jnp.float32)
    self_p = jnp.dot(feat_ref[off, :], ws0_ref[...],
                     preferred_element_type=jnp.float32)
    out0 = neigh * rd_ref[off, :] + self_p + b0_ref[...]
    rows_o = i * _T + jax.lax.broadcasted_iota(jnp.int32, out0.shape, 0)
    outb = jnp.where(rows_o < _N, out0, 0.0).astype(jnp.bfloat16)
    x_s[off, :] = outb
    x32 = outb.astype(jnp.float32)
    s_s[...] += jnp.sum(x32, axis=0, keepdims=True)
    q_s[...] += jnp.sum(x32 * x32, axis=0, keepdims=True)

    @pl.when(i == _NT - 1)
    def _():
        mean0 = s_s[...] * inv_n
        var0 = q_s[...] * inv_n - mean0 * mean0
        rstd0 = jax.lax.rsqrt(var0 + 1e-5)

        def h_hp1(t, c):
            o = pl.ds(pl.multiple_of(t * _T, _T), _T)
            x = x_s[o, :].astype(jnp.float32)
            hn = jnp.maximum((x - mean0) * rstd0, 0.0).astype(jnp.bfloat16)
            h_s[o, :] = hn
            hp_s[o, :] = jnp.dot(hn, wn1_ref[...],
                                 preferred_element_type=jnp.float32
                                 ).astype(jnp.bfloat16)
            return c
        jax.lax.fori_loop(0, _NT, h_hp1, 0)

        def agg1(t, carry):
            s, q = carry
            o = pl.ds(pl.multiple_of(t * _T, _T), _T)
            ab1 = a8_s[o, :].astype(jnp.bfloat16)
            ng = jnp.dot(ab1, hp_s[0:_N, :], preferred_element_type=jnp.float32)
            sp = jnp.dot(h_s[o, :], ws1_ref[...],
                         preferred_element_type=jnp.float32)
            o1 = ng * rd_ref[o, :] + sp + b1_ref[...]
            rows = t * _T + jax.lax.broadcasted_iota(jnp.int32, o1.shape, 0)
            ob = jnp.where(rows < _N, o1, 0.0).astype(jnp.bfloat16)
            x_s[o, :] = ob
            xf = ob.astype(jnp.float32)
            return (s + jnp.sum(xf, axis=0, keepdims=True),
                    q + jnp.sum(xf * xf, axis=0, keepdims=True))

        z = jnp.zeros((1, x_s.shape[1]), jnp.float32)
        s1, q1 = jax.lax.fori_loop(0, _NT, agg1, (z, z))
        mean1 = s1 * inv_n
        var1 = q1 * inv_n - mean1 * mean1
        rstd1 = jax.lax.rsqrt(var1 + 1e-5)

        d2 = out_ref.shape[1]

        def h_hp2(t, c):
            o = pl.ds(pl.multiple_of(t * _T, _T), _T)
            x = x_s[o, :].astype(jnp.float32)
            hn = jnp.maximum((x - mean1) * rstd1, 0.0).astype(jnp.bfloat16)
            h_s[o, :] = hn
            hp_s[o, 0:d2] = jnp.dot(hn, wn2_ref[...],
                                    preferred_element_type=jnp.float32
                                    ).astype(jnp.bfloat16)
            return c
        jax.lax.fori_loop(0, _NT, h_hp2, 0)

        def agg2(t, c):
            o = pl.ds(pl.multiple_of(t * _T, _T), _T)
            ab2 = a8_s[o, :].astype(jnp.bfloat16)
            ng = jnp.dot(ab2, hp_s[0:_N, 0:d2],
                         preferred_element_type=jnp.float32)
            sp = jnp.dot(h_s[o, :], ws2_ref[...],
                         preferred_element_type=jnp.float32)
            out_ref[o, :] = ng * rd_ref[o, :] + sp + b2_ref[...]
            return c
        jax.lax.fori_loop(0, _NT, agg2, 0)


def kernel(adj_bin, recip_deg, feat,
           w_self_0, w_neigh_0, bias_0,
           w_self_1, w_neigh_1, bias_1,
           w_self_2, w_neigh_2, bias_2):
    n, d_in = feat.shape
    d_hid = w_self_0.shape[1]
    n_cls = w_self_2.shape[1]
    d2 = _round_up(n_cls, _LANE)

    feat_p = jnp.pad(feat, ((0, _N_PAD - n), (0, _round_up(d_in, _LANE) - d_in))
                     ).astype(jnp.bfloat16)
    rd_p = jnp.pad(recip_deg, ((0, _N_PAD - n), (0, 0))).astype(jnp.float32)

    def prep_w(w):
        d_i, d_o = w.shape
        return jnp.pad(w, ((0, _round_up(d_i, _LANE) - d_i),
                           (0, _round_up(d_o, _LANE) - d_o))
                       ).astype(jnp.bfloat16)

    def prep_b(b):
        d_o = b.shape[1]
        return jnp.pad(b, ((0, 0), (0, _round_up(d_o, _LANE) - d_o))
                       ).astype(jnp.float32)

    const = lambda i: (0, 0)
    out = pl.pallas_call(
        _mega_kernel,
        out_shape=jax.ShapeDtypeStruct((_N_PAD, d2), jnp.float32),
        grid=(_NT,),
        in_specs=[
            pl.BlockSpec((_T, _N), lambda i: (i, 0)),
            pl.BlockSpec((_N_PAD, _round_up(feat.shape[1], _LANE)), const),
            pl.BlockSpec((_N_PAD, 1), const),
            pl.BlockSpec(prep_w(w_neigh_0).shape, const),
            pl.BlockSpec(prep_w(w_self_0).shape, const),
            pl.BlockSpec((1, d_hid), const),
            pl.BlockSpec(prep_w(w_neigh_1).shape, const),
            pl.BlockSpec(prep_w(w_self_1).shape, const),
            pl.BlockSpec((1, d_hid), const),
            pl.BlockSpec(prep_w(w_neigh_2).shape, const),
            pl.BlockSpec(prep_w(w_self_2).shape, const),
            pl.BlockSpec((1, d2), const),
        ],
        out_specs=pl.BlockSpec((_N_PAD, d2), const),
        scratch_shapes=[
            pltpu.VMEM((_N_PAD, _N), jnp.float8_e4m3fn),
            pltpu.VMEM((_N_PAD, d_hid), jnp.bfloat16),
            pltpu.VMEM((_N_PAD, d_hid), jnp.bfloat16),
            pltpu.VMEM((_N_PAD, d_hid), jnp.bfloat16),
            pltpu.VMEM((1, d_hid), jnp.float32),
            pltpu.VMEM((1, d_hid), jnp.float32),
        ],
        compiler_params=pltpu.CompilerParams(
            dimension_semantics=("arbitrary",),
            vmem_limit_bytes=55 * 1024 * 1024),
        cost_estimate=pl.CostEstimate(
            flops=2 * _N_PAD * _N * (2 * d_hid + d2)
            + 2 * _N_PAD * d_hid * (128 + 2 * d_hid + 2 * d2),
            transcendentals=0,
            bytes_accessed=_N * _N * 4 + _N_PAD * d2 * 4),
    )(adj_bin, feat_p, rd_p,
      prep_w(w_neigh_0), prep_w(w_self_0), prep_b(bias_0),
      prep_w(w_neigh_1), prep_w(w_self_1), prep_b(bias_1),
      prep_w(w_neigh_2), prep_w(w_self_2), prep_b(bias_2))
    return out[:n, :n_cls]
